# Initial kernel scaffold; baseline (speedup 1.0000x reference)
import functools

import jax
import jax.numpy as jnp
from jax import lax
from jax.experimental import pallas as pl
from jax.experimental.pallas import tpu as pltpu
from jax.experimental.pallas import tpu_sc as plsc

NC = 2
NS = 16
NW = NC * NS
CK = 128
BM = 256


def _sc_degree(dst_g, np_rows):
    _, ch, _ = dst_g.shape
    rpt = np_rows // NS
    mesh = plsc.VectorSubcoreMesh(core_axis_name="c", subcore_axis_name="s")

    @functools.partial(
        pl.kernel,
        out_type=jax.ShapeDtypeStruct((NC, np_rows, 16), jnp.float32),
        mesh=mesh,
        scratch_types=[
            pltpu.VMEM((ch, CK), jnp.int32),
            pltpu.VMEM((CK, 16), jnp.float32),
            pltpu.VMEM((rpt, 16), jnp.float32),
            pltpu.VMEM_SHARED((np_rows, 16), jnp.float32),
            pltpu.SemaphoreType.DMA,
        ],
    )
    def deg_kernel(dst_hbm, out_hbm, dst_v, ones_v, zbuf, acc, sem):
        cid = lax.axis_index("c")
        sid = lax.axis_index("s")
        wid = cid * NS + sid

        @pl.loop(0, rpt)
        def _(r):
            zbuf.at[pl.ds(r, 1), pl.ds(0, 16)][...] = jnp.zeros(
                (1, 16), jnp.float32)

        @pl.loop(0, CK)
        def _(r):
            ones_v.at[pl.ds(r, 1), pl.ds(0, 16)][...] = jnp.ones(
                (1, 16), jnp.float32)

        pltpu.sync_copy(zbuf, acc.at[pl.ds(sid * rpt, rpt)])
        pltpu.async_copy(dst_hbm.at[wid], dst_v, sem).wait()
        plsc.subcore_barrier()

        @pl.loop(0, ch)
        def _(j):
            pltpu.sync_copy(ones_v, acc.at[dst_v.at[j]], add=True)

        plsc.subcore_barrier()
        pltpu.sync_copy(acc.at[pl.ds(sid * rpt, rpt)],
                        out_hbm.at[cid, pl.ds(sid * rpt, rpt)])

    return deg_kernel(dst_g)


def _sc_aggregate(y, src_g, dst_g):
    np_rows, d = y.shape
    _, ch, _ = src_g.shape
    rpt = np_rows // NS
    zb = 64
    mesh = plsc.VectorSubcoreMesh(core_axis_name="c", subcore_axis_name="s")

    @functools.partial(
        pl.kernel,
        out_type=jax.ShapeDtypeStruct((NC, np_rows, d), jnp.float32),
        mesh=mesh,
        scratch_types=[
            pltpu.VMEM((ch, CK), jnp.int32),
            pltpu.VMEM((ch, CK), jnp.int32),
            pltpu.VMEM((CK, d), jnp.float32),
            pltpu.VMEM((zb, d), jnp.float32),
            pltpu.VMEM_SHARED((np_rows, d), jnp.float32),
            pltpu.SemaphoreType.DMA,
        ],
    )
    def agg_kernel(y_hbm, src_hbm, dst_hbm, out_hbm,
                   src_v, dst_v, rows_v, zbuf, acc, sem):
        cid = lax.axis_index("c")
        sid = lax.axis_index("s")
        wid = cid * NS + sid

        @pl.loop(0, zb)
        def _(r):
            @pl.loop(0, d, step=16)
            def _(c):
                zbuf.at[pl.ds(r, 1), pl.ds(c, 16)][...] = jnp.zeros(
                    (1, 16), jnp.float32)

        @pl.loop(0, rpt, step=zb)
        def _(r0):
            pltpu.sync_copy(zbuf, acc.at[pl.ds(sid * rpt + r0, zb)])

        pltpu.async_copy(src_hbm.at[wid], src_v, sem).wait()
        pltpu.async_copy(dst_hbm.at[wid], dst_v, sem).wait()
        plsc.subcore_barrier()

        @pl.loop(0, ch)
        def _(j):
            pltpu.async_copy(y_hbm.at[src_v.at[j]], rows_v, sem).wait()
            pltpu.sync_copy(rows_v, acc.at[dst_v.at[j]], add=True)

        plsc.subcore_barrier()
        pltpu.sync_copy(acc.at[pl.ds(sid * rpt, rpt)],
                        out_hbm.at[cid, pl.ds(sid * rpt, rpt)])

    return agg_kernel(y, src_g, dst_g)


def _mm_body(x_ref, w_ref, o_ref):
    o_ref[...] = jnp.dot(x_ref[...], w_ref[...],
                         preferred_element_type=jnp.float32)


def _tc_matmul(x, w):
    np_rows, k = x.shape
    dout = w.shape[1]
    return pl.pallas_call(
        _mm_body,
        grid=(np_rows // BM,),
        in_specs=[pl.BlockSpec((BM, k), lambda i: (i, 0)),
                  pl.BlockSpec((k, dout), lambda i: (0, 0))],
        out_specs=pl.BlockSpec((BM, dout), lambda i: (i, 0)),
        out_shape=jax.ShapeDtypeStruct((np_rows, dout), jnp.float32),
    )(x, w)


def _tc_scale1(xw, degacc, n_real):
    np_rows, d = xw.shape

    def body(xw_ref, dacc_ref, y_ref, s_ref, dinv_ref):
        i = pl.program_id(0)
        deg = dacc_ref[0][:, 0:1] + dacc_ref[1][:, 0:1] + 1.0
        dinv = lax.rsqrt(deg)
        row = lax.broadcasted_iota(jnp.int32, (BM, 1), 0) + i * BM
        dinv = jnp.where(row < n_real, dinv, 0.0)
        y = xw_ref[...] * dinv
        y_ref[...] = y
        s_ref[...] = y * dinv
        dinv_ref[...] = dinv

    return pl.pallas_call(
        body,
        grid=(np_rows // BM,),
        in_specs=[pl.BlockSpec((BM, d), lambda i: (i, 0)),
                  pl.BlockSpec((NC, BM, 16), lambda i: (0, i, 0))],
        out_specs=[pl.BlockSpec((BM, d), lambda i: (i, 0)),
                   pl.BlockSpec((BM, d), lambda i: (i, 0)),
                   pl.BlockSpec((BM, 1), lambda i: (i, 0))],
        out_shape=[jax.ShapeDtypeStruct((np_rows, d), jnp.float32),
                   jax.ShapeDtypeStruct((np_rows, d), jnp.float32),
                   jax.ShapeDtypeStruct((np_rows, 1), jnp.float32)],
    )(xw, degacc)


def _tc_mid(agg, self1, dinv, b1, w2, n_real):
    _, np_rows, d = agg.shape
    d2 = w2.shape[1]

    def body(agg_ref, s1_ref, dinv_ref, b1_ref, w2_ref, y2_ref, s2_ref):
        i = pl.program_id(0)
        dv = dinv_ref[...]
        h = (agg_ref[0] + agg_ref[1]) * dv + s1_ref[...] + b1_ref[...]
        h = jnp.maximum(h, 0.0)
        row = lax.broadcasted_iota(jnp.int32, (BM, 1), 0) + i * BM
        h = jnp.where(row < n_real, h, 0.0)
        xw2 = jnp.dot(h, w2_ref[...], preferred_element_type=jnp.float32)
        y2 = xw2 * dv
        y2_ref[...] = y2
        s2_ref[...] = y2 * dv

    return pl.pallas_call(
        body,
        grid=(np_rows // BM,),
        in_specs=[pl.BlockSpec((NC, BM, d), lambda i: (0, i, 0)),
                  pl.BlockSpec((BM, d), lambda i: (i, 0)),
                  pl.BlockSpec((BM, 1), lambda i: (i, 0)),
                  pl.BlockSpec((1, d), lambda i: (0, 0)),
                  pl.BlockSpec((d, d2), lambda i: (0, 0))],
        out_specs=[pl.BlockSpec((BM, d2), lambda i: (i, 0)),
                   pl.BlockSpec((BM, d2), lambda i: (i, 0))],
        out_shape=[jax.ShapeDtypeStruct((np_rows, d2), jnp.float32),
                   jax.ShapeDtypeStruct((np_rows, d2), jnp.float32)],
    )(agg, self1, dinv, b1, w2)


def _tc_final(agg, self2, dinv, b2):
    _, np_rows, d2 = agg.shape

    def body(agg_ref, s2_ref, dinv_ref, b2_ref, o_ref):
        o = ((agg_ref[0] + agg_ref[1]) * dinv_ref[...]
             + s2_ref[...] + b2_ref[...])
        m = jnp.max(o, axis=1, keepdims=True)
        z = o - m
        lse = jnp.log(jnp.sum(jnp.exp(z), axis=1, keepdims=True))
        o_ref[...] = z - lse

    return pl.pallas_call(
        body,
        grid=(np_rows // BM,),
        in_specs=[pl.BlockSpec((NC, BM, d2), lambda i: (0, i, 0)),
                  pl.BlockSpec((BM, d2), lambda i: (i, 0)),
                  pl.BlockSpec((BM, 1), lambda i: (i, 0)),
                  pl.BlockSpec((1, d2), lambda i: (0, 0))],
        out_specs=pl.BlockSpec((BM, d2), lambda i: (i, 0)),
        out_shape=jax.ShapeDtypeStruct((np_rows, d2), jnp.float32),
    )(agg, self2, dinv, b2)


def kernel(x, edge_index, W1, b1, W2, b2):
    n, d1 = x.shape
    e = edge_index.shape[1]
    ei = edge_index.astype(jnp.int32)

    ep = -(-e // (NW * CK)) * (NW * CK)
    pad_e = ep - e
    src = jnp.concatenate([ei[0], jnp.full((pad_e,), n, jnp.int32)])
    dst = jnp.concatenate([ei[1], jnp.full((pad_e,), n, jnp.int32)])
    ch = ep // (NW * CK)
    src_g = src.reshape(NW, ch, CK)
    dst_g = dst.reshape(NW, ch, CK)

    np_rows = -(-(n + 1) // 1024) * 1024
    x_pad = jnp.concatenate([x, jnp.zeros((np_rows - n, d1), x.dtype)])

    degacc = _sc_degree(dst_g, np_rows)
    xw1 = _tc_matmul(x_pad, W1)
    y1, self1, dinv = _tc_scale1(xw1, degacc, n)
    agg1 = _sc_aggregate(y1, src_g, dst_g)
    y2, self2 = _tc_mid(agg1, self1, dinv, b1.reshape(1, -1), W2, n)
    agg2 = _sc_aggregate(y2, src_g, dst_g)
    out = _tc_final(agg2, self2, dinv, b2.reshape(1, -1))
    return out[:n]


# speedup vs baseline: 14.7406x; 14.7406x over previous
import functools

import jax
import jax.numpy as jnp
from jax import lax
from jax.experimental import pallas as pl
from jax.experimental.pallas import tpu as pltpu
from jax.experimental.pallas import tpu_sc as plsc

NC = 2
NS = 16
NW = NC * NS
CK = 128
BM = 256


def _sc_degree(dst_g, np_rows):
    _, ch, _ = dst_g.shape
    rpt = np_rows // NS
    mesh = plsc.VectorSubcoreMesh(core_axis_name="c", subcore_axis_name="s", num_cores=NC, num_subcores=NS)

    @functools.partial(
        pl.kernel,
        out_type=jax.ShapeDtypeStruct((NC, np_rows, 16), jnp.float32),
        mesh=mesh,
        compiler_params=pltpu.CompilerParams(use_tc_tiling_on_sc=False),
        scratch_types=[
            pltpu.VMEM((ch, CK), jnp.int32),
            pltpu.VMEM((CK, 16), jnp.float32),
            pltpu.VMEM((rpt, 16), jnp.float32),
            pltpu.VMEM_SHARED((np_rows, 16), jnp.float32),
            pltpu.SemaphoreType.DMA,
        ],
    )
    def deg_kernel(dst_hbm, out_hbm, dst_v, ones_v, zbuf, acc, sem):
        cid = lax.axis_index("c")
        sid = lax.axis_index("s")
        wid = cid * NS + sid

        @pl.loop(0, rpt)
        def _(r):
            zbuf.at[pl.ds(r, 1), pl.ds(0, 16)][...] = jnp.zeros(
                (1, 16), jnp.float32)

        @pl.loop(0, CK)
        def _(r):
            ones_v.at[pl.ds(r, 1), pl.ds(0, 16)][...] = jnp.ones(
                (1, 16), jnp.float32)

        pltpu.sync_copy(zbuf, acc.at[pl.ds(sid * rpt, rpt)])
        pltpu.async_copy(dst_hbm.at[wid], dst_v, sem).wait()
        plsc.subcore_barrier()

        @pl.loop(0, ch)
        def _(j):
            pltpu.sync_copy(ones_v, acc.at[dst_v.at[j]], add=True)

        plsc.subcore_barrier()
        pltpu.sync_copy(acc.at[pl.ds(sid * rpt, rpt)],
                        out_hbm.at[cid, pl.ds(sid * rpt, rpt)])

    return deg_kernel(dst_g)


def _sc_aggregate(y, src_g, dst_g):
    np_rows, d = y.shape
    _, ch, _ = src_g.shape
    rpt = np_rows // NS
    zb = 64
    mesh = plsc.VectorSubcoreMesh(core_axis_name="c", subcore_axis_name="s", num_cores=NC, num_subcores=NS)

    @functools.partial(
        pl.kernel,
        out_type=jax.ShapeDtypeStruct((NC, np_rows, d), jnp.float32),
        mesh=mesh,
        compiler_params=pltpu.CompilerParams(use_tc_tiling_on_sc=False),
        scratch_types=[
            pltpu.VMEM((ch, CK), jnp.int32),
            pltpu.VMEM((ch, CK), jnp.int32),
            pltpu.VMEM((CK, d), jnp.float32),
            pltpu.VMEM((zb, d), jnp.float32),
            pltpu.VMEM_SHARED((np_rows, d), jnp.float32),
            pltpu.SemaphoreType.DMA,
        ],
    )
    def agg_kernel(y_hbm, src_hbm, dst_hbm, out_hbm,
                   src_v, dst_v, rows_v, zbuf, acc, sem):
        cid = lax.axis_index("c")
        sid = lax.axis_index("s")
        wid = cid * NS + sid

        @pl.loop(0, zb)
        def _(r):
            @pl.loop(0, d, step=16)
            def _(c):
                zbuf.at[pl.ds(r, 1), pl.ds(c, 16)][...] = jnp.zeros(
                    (1, 16), jnp.float32)

        @pl.loop(0, rpt, step=zb)
        def _(r0):
            pltpu.sync_copy(zbuf, acc.at[pl.ds(sid * rpt + r0, zb)])

        pltpu.async_copy(src_hbm.at[wid], src_v, sem).wait()
        pltpu.async_copy(dst_hbm.at[wid], dst_v, sem).wait()
        plsc.subcore_barrier()

        @pl.loop(0, ch)
        def _(j):
            pltpu.async_copy(y_hbm.at[src_v.at[j]], rows_v, sem).wait()
            pltpu.sync_copy(rows_v, acc.at[dst_v.at[j]], add=True)

        plsc.subcore_barrier()
        pltpu.sync_copy(acc.at[pl.ds(sid * rpt, rpt)],
                        out_hbm.at[cid, pl.ds(sid * rpt, rpt)])

    return agg_kernel(y, src_g, dst_g)


def _mm_body(x_ref, w_ref, o_ref):
    o_ref[...] = jnp.dot(x_ref[...], w_ref[...],
                         preferred_element_type=jnp.float32)


def _tc_matmul(x, w):
    np_rows, k = x.shape
    dout = w.shape[1]
    return pl.pallas_call(
        _mm_body,
        grid=(np_rows // BM,),
        in_specs=[pl.BlockSpec((BM, k), lambda i: (i, 0)),
                  pl.BlockSpec((k, dout), lambda i: (0, 0))],
        out_specs=pl.BlockSpec((BM, dout), lambda i: (i, 0)),
        out_shape=jax.ShapeDtypeStruct((np_rows, dout), jnp.float32),
    )(x, w)


def _tc_scale1(xw, degacc, n_real):
    np_rows, d = xw.shape

    def body(xw_ref, dacc_ref, y_ref, s_ref, dinv_ref):
        i = pl.program_id(0)
        deg = dacc_ref[0][:, 0:1] + dacc_ref[1][:, 0:1] + 1.0
        dinv = lax.rsqrt(deg)
        row = lax.broadcasted_iota(jnp.int32, (BM, 1), 0) + i * BM
        dinv = jnp.where(row < n_real, dinv, 0.0)
        y = xw_ref[...] * dinv
        y_ref[...] = y
        s_ref[...] = y * dinv
        dinv_ref[...] = dinv

    return pl.pallas_call(
        body,
        grid=(np_rows // BM,),
        in_specs=[pl.BlockSpec((BM, d), lambda i: (i, 0)),
                  pl.BlockSpec((NC, BM, 16), lambda i: (0, i, 0))],
        out_specs=[pl.BlockSpec((BM, d), lambda i: (i, 0)),
                   pl.BlockSpec((BM, d), lambda i: (i, 0)),
                   pl.BlockSpec((BM, 1), lambda i: (i, 0))],
        out_shape=[jax.ShapeDtypeStruct((np_rows, d), jnp.float32),
                   jax.ShapeDtypeStruct((np_rows, d), jnp.float32),
                   jax.ShapeDtypeStruct((np_rows, 1), jnp.float32)],
    )(xw, degacc)


def _tc_mid(agg, self1, dinv, b1, w2, n_real):
    _, np_rows, d = agg.shape
    d2 = w2.shape[1]

    def body(agg_ref, s1_ref, dinv_ref, b1_ref, w2_ref, y2_ref, s2_ref):
        i = pl.program_id(0)
        dv = dinv_ref[...]
        h = (agg_ref[0] + agg_ref[1]) * dv + s1_ref[...] + b1_ref[...]
        h = jnp.maximum(h, 0.0)
        row = lax.broadcasted_iota(jnp.int32, (BM, 1), 0) + i * BM
        h = jnp.where(row < n_real, h, 0.0)
        xw2 = jnp.dot(h, w2_ref[...], preferred_element_type=jnp.float32)
        y2 = xw2 * dv
        y2_ref[...] = y2
        s2_ref[...] = y2 * dv

    return pl.pallas_call(
        body,
        grid=(np_rows // BM,),
        in_specs=[pl.BlockSpec((NC, BM, d), lambda i: (0, i, 0)),
                  pl.BlockSpec((BM, d), lambda i: (i, 0)),
                  pl.BlockSpec((BM, 1), lambda i: (i, 0)),
                  pl.BlockSpec((1, d), lambda i: (0, 0)),
                  pl.BlockSpec((d, d2), lambda i: (0, 0))],
        out_specs=[pl.BlockSpec((BM, d2), lambda i: (i, 0)),
                   pl.BlockSpec((BM, d2), lambda i: (i, 0))],
        out_shape=[jax.ShapeDtypeStruct((np_rows, d2), jnp.float32),
                   jax.ShapeDtypeStruct((np_rows, d2), jnp.float32)],
    )(agg, self1, dinv, b1, w2)


def _tc_final(agg, self2, dinv, b2):
    _, np_rows, d2 = agg.shape

    def body(agg_ref, s2_ref, dinv_ref, b2_ref, o_ref):
        o = ((agg_ref[0] + agg_ref[1]) * dinv_ref[...]
             + s2_ref[...] + b2_ref[...])
        m = jnp.max(o, axis=1, keepdims=True)
        z = o - m
        lse = jnp.log(jnp.sum(jnp.exp(z), axis=1, keepdims=True))
        o_ref[...] = z - lse

    return pl.pallas_call(
        body,
        grid=(np_rows // BM,),
        in_specs=[pl.BlockSpec((NC, BM, d2), lambda i: (0, i, 0)),
                  pl.BlockSpec((BM, d2), lambda i: (i, 0)),
                  pl.BlockSpec((BM, 1), lambda i: (i, 0)),
                  pl.BlockSpec((1, d2), lambda i: (0, 0))],
        out_specs=pl.BlockSpec((BM, d2), lambda i: (i, 0)),
        out_shape=jax.ShapeDtypeStruct((np_rows, d2), jnp.float32),
    )(agg, self2, dinv, b2)


def kernel(x, edge_index, W1, b1, W2, b2):
    n, d1 = x.shape
    e = edge_index.shape[1]
    ei = edge_index.astype(jnp.int32)

    ep = -(-e // (NW * CK)) * (NW * CK)
    pad_e = ep - e
    src = jnp.concatenate([ei[0], jnp.full((pad_e,), n, jnp.int32)])
    dst = jnp.concatenate([ei[1], jnp.full((pad_e,), n, jnp.int32)])
    ch = ep // (NW * CK)
    src_g = src.reshape(NW, ch, CK)
    dst_g = dst.reshape(NW, ch, CK)

    np_rows = -(-(n + 1) // 1024) * 1024
    x_pad = jnp.concatenate([x, jnp.zeros((np_rows - n, d1), x.dtype)])

    degacc = _sc_degree(dst_g, np_rows)
    xw1 = _tc_matmul(x_pad, W1)
    y1, self1, dinv = _tc_scale1(xw1, degacc, n)
    agg1 = _sc_aggregate(y1, src_g, dst_g)
    y2, self2 = _tc_mid(agg1, self1, dinv, b1.reshape(1, -1), W2, n)
    agg2 = _sc_aggregate(y2, src_g, dst_g)
    out = _tc_final(agg2, self2, dinv, b2.reshape(1, -1))
    return out[:n]


# speedup vs baseline: 16.3127x; 1.1066x over previous
import functools

import jax
import jax.numpy as jnp
from jax import lax
from jax.experimental import pallas as pl
from jax.experimental.pallas import tpu as pltpu
from jax.experimental.pallas import tpu_sc as plsc

NC = 2
NS = 16
NW = NC * NS
CK = 128
BM = 256


def _sc_degree(dst_g, np_rows):
    _, ch, _ = dst_g.shape
    rpt = np_rows // NS
    mesh = plsc.VectorSubcoreMesh(core_axis_name="c", subcore_axis_name="s", num_cores=NC, num_subcores=NS)

    @functools.partial(
        pl.kernel,
        out_type=jax.ShapeDtypeStruct((NC, np_rows, 16), jnp.float32),
        mesh=mesh,
        compiler_params=pltpu.CompilerParams(use_tc_tiling_on_sc=False),
        scratch_types=[
            pltpu.VMEM((ch, CK), jnp.int32),
            pltpu.VMEM((CK, 16), jnp.float32),
            pltpu.VMEM((rpt, 16), jnp.float32),
            pltpu.VMEM_SHARED((np_rows, 16), jnp.float32),
            pltpu.SemaphoreType.DMA,
        ],
    )
    def deg_kernel(dst_hbm, out_hbm, dst_v, ones_v, zbuf, acc, sem):
        cid = lax.axis_index("c")
        sid = lax.axis_index("s")
        wid = cid * NS + sid

        @pl.loop(0, rpt)
        def _(r):
            zbuf.at[pl.ds(r, 1), pl.ds(0, 16)][...] = jnp.zeros(
                (1, 16), jnp.float32)

        @pl.loop(0, CK)
        def _(r):
            ones_v.at[pl.ds(r, 1), pl.ds(0, 16)][...] = jnp.ones(
                (1, 16), jnp.float32)

        pltpu.sync_copy(zbuf, acc.at[pl.ds(sid * rpt, rpt)])
        pltpu.async_copy(dst_hbm.at[wid], dst_v, sem).wait()
        plsc.subcore_barrier()

        @pl.loop(0, ch)
        def _(j):
            pltpu.sync_copy(ones_v, acc.at[dst_v.at[j]], add=True)

        plsc.subcore_barrier()
        pltpu.sync_copy(acc.at[pl.ds(sid * rpt, rpt)],
                        out_hbm.at[cid, pl.ds(sid * rpt, rpt)])

    return deg_kernel(dst_g)


def _sc_aggregate(y, src_g, dst_g):
    np_rows, d = y.shape
    _, ch, _ = src_g.shape
    assert ch % 2 == 1, "double-buffered loop expects an odd chunk count"
    rpt = np_rows // NS
    zb = 16
    mesh = plsc.VectorSubcoreMesh(core_axis_name="c", subcore_axis_name="s", num_cores=NC, num_subcores=NS)

    @functools.partial(
        pl.kernel,
        out_type=jax.ShapeDtypeStruct((NC, np_rows, d), jnp.float32),
        mesh=mesh,
        compiler_params=pltpu.CompilerParams(use_tc_tiling_on_sc=False),
        scratch_types=[
            pltpu.VMEM((ch, CK), jnp.int32),
            pltpu.VMEM((2, CK), jnp.int32),
            pltpu.VMEM((CK, d), jnp.float32),
            pltpu.VMEM((CK, d), jnp.float32),
            pltpu.VMEM((zb, d), jnp.float32),
            pltpu.VMEM_SHARED((np_rows, d), jnp.float32),
            pltpu.SemaphoreType.DMA,
            pltpu.SemaphoreType.DMA,
            pltpu.SemaphoreType.DMA,
        ],
    )
    def agg_kernel(y_hbm, src_hbm, dst_hbm, out_hbm,
                   src_v, dst_v, rows0_v, rows1_v, zbuf, acc,
                   sem0, sem1, semi):
        cid = lax.axis_index("c")
        sid = lax.axis_index("s")
        wid = cid * NS + sid

        @pl.loop(0, zb)
        def _(r):
            @pl.loop(0, d, step=16)
            def _(c):
                zbuf.at[pl.ds(r, 1), pl.ds(c, 16)][...] = jnp.zeros(
                    (1, 16), jnp.float32)

        @pl.loop(0, rpt, step=zb)
        def _(r0):
            pltpu.sync_copy(zbuf, acc.at[pl.ds(sid * rpt + r0, zb)])

        pltpu.async_copy(src_hbm.at[wid], src_v, sem0).wait()
        plsc.subcore_barrier()

        pltpu.sync_copy(dst_hbm.at[wid, pl.ds(0, 1)], dst_v.at[pl.ds(0, 1)])
        pltpu.async_copy(y_hbm.at[src_v.at[0]], rows0_v, sem0)

        @pl.loop(0, ch - 1, step=2)
        def _(j):
            pltpu.async_copy(dst_hbm.at[wid, pl.ds(j + 1, 1)],
                             dst_v.at[pl.ds(1, 1)], semi)
            pltpu.make_async_copy(y_hbm.at[src_v.at[j]], rows0_v, sem0).wait()
            pltpu.async_copy(y_hbm.at[src_v.at[j + 1]], rows1_v, sem1)
            pltpu.sync_copy(rows0_v, acc.at[dst_v.at[0]], add=True)
            pltpu.make_async_copy(dst_hbm.at[wid, pl.ds(j + 1, 1)],
                                  dst_v.at[pl.ds(1, 1)], semi).wait()
            pltpu.async_copy(dst_hbm.at[wid, pl.ds(j + 2, 1)],
                             dst_v.at[pl.ds(0, 1)], semi)
            pltpu.make_async_copy(
                y_hbm.at[src_v.at[j + 1]], rows1_v, sem1).wait()
            pltpu.async_copy(y_hbm.at[src_v.at[j + 2]], rows0_v, sem0)
            pltpu.sync_copy(rows1_v, acc.at[dst_v.at[1]], add=True)
            pltpu.make_async_copy(dst_hbm.at[wid, pl.ds(j + 2, 1)],
                                  dst_v.at[pl.ds(0, 1)], semi).wait()

        pltpu.make_async_copy(
            y_hbm.at[src_v.at[ch - 1]], rows0_v, sem0).wait()
        pltpu.sync_copy(rows0_v, acc.at[dst_v.at[0]], add=True)

        plsc.subcore_barrier()
        pltpu.sync_copy(acc.at[pl.ds(sid * rpt, rpt)],
                        out_hbm.at[cid, pl.ds(sid * rpt, rpt)])

    return agg_kernel(y, src_g, dst_g)


def _mm_body(x_ref, w_ref, o_ref):
    o_ref[...] = jnp.dot(x_ref[...], w_ref[...],
                         preferred_element_type=jnp.float32)


def _tc_matmul(x, w):
    np_rows, k = x.shape
    dout = w.shape[1]
    return pl.pallas_call(
        _mm_body,
        grid=(np_rows // BM,),
        in_specs=[pl.BlockSpec((BM, k), lambda i: (i, 0)),
                  pl.BlockSpec((k, dout), lambda i: (0, 0))],
        out_specs=pl.BlockSpec((BM, dout), lambda i: (i, 0)),
        out_shape=jax.ShapeDtypeStruct((np_rows, dout), jnp.float32),
    )(x, w)


def _tc_scale1(xw, degacc, n_real):
    np_rows, d = xw.shape

    def body(xw_ref, dacc_ref, y_ref, s_ref, dinv_ref):
        i = pl.program_id(0)
        deg = dacc_ref[0][:, 0:1] + dacc_ref[1][:, 0:1] + 1.0
        dinv = lax.rsqrt(deg)
        row = lax.broadcasted_iota(jnp.int32, (BM, 1), 0) + i * BM
        dinv = jnp.where(row < n_real, dinv, 0.0)
        y = xw_ref[...] * dinv
        y_ref[...] = y
        s_ref[...] = y * dinv
        dinv_ref[...] = dinv

    return pl.pallas_call(
        body,
        grid=(np_rows // BM,),
        in_specs=[pl.BlockSpec((BM, d), lambda i: (i, 0)),
                  pl.BlockSpec((NC, BM, 16), lambda i: (0, i, 0))],
        out_specs=[pl.BlockSpec((BM, d), lambda i: (i, 0)),
                   pl.BlockSpec((BM, d), lambda i: (i, 0)),
                   pl.BlockSpec((BM, 1), lambda i: (i, 0))],
        out_shape=[jax.ShapeDtypeStruct((np_rows, d), jnp.float32),
                   jax.ShapeDtypeStruct((np_rows, d), jnp.float32),
                   jax.ShapeDtypeStruct((np_rows, 1), jnp.float32)],
    )(xw, degacc)


def _tc_mid(agg, self1, dinv, b1, w2, n_real):
    _, np_rows, d = agg.shape
    d2 = w2.shape[1]

    def body(agg_ref, s1_ref, dinv_ref, b1_ref, w2_ref, y2_ref, s2_ref):
        i = pl.program_id(0)
        dv = dinv_ref[...]
        h = (agg_ref[0] + agg_ref[1]) * dv + s1_ref[...] + b1_ref[...]
        h = jnp.maximum(h, 0.0)
        row = lax.broadcasted_iota(jnp.int32, (BM, 1), 0) + i * BM
        h = jnp.where(row < n_real, h, 0.0)
        xw2 = jnp.dot(h, w2_ref[...], preferred_element_type=jnp.float32)
        y2 = xw2 * dv
        y2_ref[...] = y2
        s2_ref[...] = y2 * dv

    return pl.pallas_call(
        body,
        grid=(np_rows // BM,),
        in_specs=[pl.BlockSpec((NC, BM, d), lambda i: (0, i, 0)),
                  pl.BlockSpec((BM, d), lambda i: (i, 0)),
                  pl.BlockSpec((BM, 1), lambda i: (i, 0)),
                  pl.BlockSpec((1, d), lambda i: (0, 0)),
                  pl.BlockSpec((d, d2), lambda i: (0, 0))],
        out_specs=[pl.BlockSpec((BM, d2), lambda i: (i, 0)),
                   pl.BlockSpec((BM, d2), lambda i: (i, 0))],
        out_shape=[jax.ShapeDtypeStruct((np_rows, d2), jnp.float32),
                   jax.ShapeDtypeStruct((np_rows, d2), jnp.float32)],
    )(agg, self1, dinv, b1, w2)


def _tc_final(agg, self2, dinv, b2):
    _, np_rows, d2 = agg.shape

    def body(agg_ref, s2_ref, dinv_ref, b2_ref, o_ref):
        o = ((agg_ref[0] + agg_ref[1]) * dinv_ref[...]
             + s2_ref[...] + b2_ref[...])
        m = jnp.max(o, axis=1, keepdims=True)
        z = o - m
        lse = jnp.log(jnp.sum(jnp.exp(z), axis=1, keepdims=True))
        o_ref[...] = z - lse

    return pl.pallas_call(
        body,
        grid=(np_rows // BM,),
        in_specs=[pl.BlockSpec((NC, BM, d2), lambda i: (0, i, 0)),
                  pl.BlockSpec((BM, d2), lambda i: (i, 0)),
                  pl.BlockSpec((BM, 1), lambda i: (i, 0)),
                  pl.BlockSpec((1, d2), lambda i: (0, 0))],
        out_specs=pl.BlockSpec((BM, d2), lambda i: (i, 0)),
        out_shape=jax.ShapeDtypeStruct((np_rows, d2), jnp.float32),
    )(agg, self2, dinv, b2)


def kernel(x, edge_index, W1, b1, W2, b2):
    n, d1 = x.shape
    e = edge_index.shape[1]
    ei = edge_index.astype(jnp.int32)

    ep = -(-e // (NW * CK)) * (NW * CK)
    pad_e = ep - e
    src = jnp.concatenate([ei[0], jnp.full((pad_e,), n, jnp.int32)])
    dst = jnp.concatenate([ei[1], jnp.full((pad_e,), n, jnp.int32)])
    ch = ep // (NW * CK)
    src_g = src.reshape(NW, ch, CK)
    dst_g = dst.reshape(NW, ch, CK)

    np_rows = -(-(n + 1) // 1024) * 1024
    x_pad = jnp.concatenate([x, jnp.zeros((np_rows - n, d1), x.dtype)])

    degacc = _sc_degree(dst_g, np_rows)
    xw1 = _tc_matmul(x_pad, W1)
    y1, self1, dinv = _tc_scale1(xw1, degacc, n)
    agg1 = _sc_aggregate(y1, src_g, dst_g)
    y2, self2 = _tc_mid(agg1, self1, dinv, b1.reshape(1, -1), W2, n)
    agg2 = _sc_aggregate(y2, src_g, dst_g)
    out = _tc_final(agg2, self2, dinv, b2.reshape(1, -1))
    return out[:n]
